# baseline (speedup 1.0000x reference)
import functools
import jax
import jax.numpy as jnp
from jax import lax
from jax.experimental import pallas as pl
from jax.experimental.pallas import tpu as pltpu, tpu_sc as plsc

D_MODEL = 1024

_info = plsc.get_sparse_core_info()
_NC, _NS = _info.num_cores, _info.num_subcores
_NW = _NC * _NS


def _make_lookup(B: int, D: int, chunk: int):
    b_per_w = B // _NW
    n_chunks = b_per_w // chunk
    mesh = plsc.VectorSubcoreMesh(core_axis_name="c", subcore_axis_name="s")

    @functools.partial(
        pl.kernel,
        mesh=mesh,
        out_type=jax.ShapeDtypeStruct((B, D), jnp.float32),
        scratch_types=[
            pltpu.VMEM((b_per_w,), jnp.int32),
            pltpu.VMEM((chunk, D), jnp.float32),
            pltpu.SemaphoreType.DMA,
        ],
    )
    def lookup(table_hbm, idx_hbm, out_hbm, idx_v, rows_v, sem):
        wid = lax.axis_index("s") * _NC + lax.axis_index("c")
        base = wid * b_per_w
        pltpu.sync_copy(idx_hbm.at[pl.ds(base, b_per_w)], idx_v)
        for c in range(n_chunks):
            gather = pltpu.async_copy(
                table_hbm.at[idx_v.at[pl.ds(c * chunk, chunk)]], rows_v, sem
            )
            gather.wait()
            pltpu.sync_copy(rows_v, out_hbm.at[pl.ds(base + c * chunk, chunk)])

    return lookup


def kernel(placeholder, table):
    B = placeholder.size
    idx = placeholder.reshape(B).astype(jnp.int32)
    out = _make_lookup(B, D_MODEL, chunk=64)(table, idx)
    return out.reshape(*placeholder.shape, D_MODEL)


# speedup vs baseline: 1.0158x; 1.0158x over previous
import functools
import jax
import jax.numpy as jnp
from jax import lax
from jax.experimental import pallas as pl
from jax.experimental.pallas import tpu as pltpu, tpu_sc as plsc

D_MODEL = 1024
NBUF = 2

_info = plsc.get_sparse_core_info()
_NC, _NS = _info.num_cores, _info.num_subcores
_NW = _NC * _NS


def _make_lookup(B: int, D: int, chunk: int):
    b_per_w = B // _NW
    n_chunks = b_per_w // chunk
    assert n_chunks % NBUF == 0 and n_chunks >= 2 * NBUF
    mesh = plsc.VectorSubcoreMesh(core_axis_name="c", subcore_axis_name="s")

    @functools.partial(
        pl.kernel,
        mesh=mesh,
        out_type=jax.ShapeDtypeStruct((B, D), jnp.float32),
        scratch_types=[
            pltpu.VMEM((b_per_w,), jnp.int32),
            pltpu.VMEM((chunk, D), jnp.float32),
            pltpu.VMEM((chunk, D), jnp.float32),
            pltpu.SemaphoreType.DMA,
            pltpu.SemaphoreType.DMA,
            pltpu.SemaphoreType.DMA,
            pltpu.SemaphoreType.DMA,
        ],
    )
    def lookup(table_hbm, idx_hbm, out_hbm, idx_v, buf0, buf1, g0, g1, s0, s1):
        wid = lax.axis_index("s") * _NC + lax.axis_index("c")
        base = wid * b_per_w
        pltpu.sync_copy(idx_hbm.at[pl.ds(base, b_per_w)], idx_v)
        bufs = (buf0, buf1)
        gsem = (g0, g1)
        ssem = (s0, s1)

        def gather(c, b):
            pltpu.async_copy(
                table_hbm.at[idx_v.at[pl.ds(c * chunk, chunk)]], bufs[b], gsem[b]
            )

        def store(c, b):
            pltpu.async_copy(
                bufs[b], out_hbm.at[pl.ds(base + c * chunk, chunk)], ssem[b]
            )

        def drain(sem, b):
            pltpu.make_async_copy(out_hbm.at[pl.ds(0, chunk)], bufs[b], sem[b]).wait()

        for b in range(NBUF):
            gather(b, b)

        @pl.loop(0, n_chunks, step=NBUF)
        def _(c0):
            for b in range(NBUF):
                c = c0 + b
                drain(gsem, b)
                store(c, b)

                @pl.when(c0 < n_chunks - NBUF)
                def _():
                    drain(ssem, b)
                    gather(c + NBUF, b)

        for b in range(NBUF):
            drain(ssem, b)

    return lookup


def kernel(placeholder, table):
    B = placeholder.size
    idx = placeholder.reshape(B).astype(jnp.int32)
    out = _make_lookup(B, D_MODEL, chunk=32)(table, idx)
    return out.reshape(*placeholder.shape, D_MODEL)


# speedup vs baseline: 1.3426x; 1.3216x over previous
import functools
import jax
import jax.numpy as jnp
from jax import lax
from jax.experimental import pallas as pl
from jax.experimental.pallas import tpu as pltpu, tpu_sc as plsc

D_MODEL = 1024
NBUF = 2
LANES = 16

_info = plsc.get_sparse_core_info()
_NC, _NS = _info.num_cores, _info.num_subcores
_NW = _NC * _NS


def _make_lookup(B: int, V: int, D: int, chunk: int):
    b_per_w = B // _NW
    n_chunks = b_per_w // chunk
    assert n_chunks % NBUF == 0 and n_chunks >= 2 * NBUF
    mesh = plsc.VectorSubcoreMesh(core_axis_name="c", subcore_axis_name="s")

    @functools.partial(
        pl.kernel,
        mesh=mesh,
        compiler_params=pltpu.CompilerParams(needs_layout_passes=False),
        out_type=jax.ShapeDtypeStruct((B, D), jnp.float32),
        scratch_types=[
            pltpu.VMEM((V * D,), jnp.float32),
            pltpu.VMEM((b_per_w,), jnp.int32),
            pltpu.VMEM((chunk, D), jnp.float32),
            pltpu.VMEM((chunk, D), jnp.float32),
            pltpu.SemaphoreType.DMA,
            pltpu.SemaphoreType.DMA,
        ],
    )
    def lookup(table_hbm, idx_hbm, out_hbm, table_v, idx_v, buf0, buf1, s0, s1):
        wid = lax.axis_index("s") * _NC + lax.axis_index("c")
        base = wid * b_per_w
        pltpu.sync_copy(table_hbm, table_v)
        pltpu.sync_copy(idx_hbm.at[pl.ds(base, b_per_w)], idx_v)
        bufs = (buf0, buf1)
        ssem = (s0, s1)

        lane = lax.iota(jnp.int32, LANES)

        def expand(c, b):
            buf = bufs[b]
            for g in range(chunk // LANES):
                idxvec = idx_v[pl.ds(c * chunk + g * LANES, LANES)]
                basevec = idxvec * D

                @pl.loop(0, LANES)
                def _(j):
                    basesplat = basevec.at[jnp.full((LANES,), j, jnp.int32)].get(
                        mode="promise_in_bounds"
                    )
                    r = g * LANES + j
                    for d0 in range(0, D, LANES):
                        addr = basesplat + (lane + d0)
                        buf[r, pl.ds(d0, LANES)] = plsc.load_gather(
                            table_v, [addr]
                        )

        def store(c, b):
            pltpu.async_copy(
                bufs[b], out_hbm.at[pl.ds(base + c * chunk, chunk)], ssem[b]
            )

        def drain(b):
            pltpu.make_async_copy(out_hbm.at[pl.ds(0, chunk)], bufs[b], ssem[b]).wait()

        @pl.loop(0, n_chunks, step=NBUF)
        def _(c0):
            for b in range(NBUF):
                c = c0 + b

                @pl.when(c0 > 0)
                def _():
                    drain(b)

                expand(c, b)
                store(c, b)

        for b in range(NBUF):
            drain(b)

    return lookup


def kernel(placeholder, table):
    B = placeholder.size
    V, D = table.shape
    idx = placeholder.reshape(B).astype(jnp.int32)
    out = _make_lookup(B, V, D, chunk=32)(table.reshape(-1), idx)
    return out.reshape(*placeholder.shape, D)


# speedup vs baseline: 4.8303x; 3.5977x over previous
import functools
import jax
import jax.numpy as jnp
from jax import lax
from jax.experimental import pallas as pl
from jax.experimental.pallas import tpu as pltpu, tpu_sc as plsc

D_MODEL = 1024

_info = plsc.get_sparse_core_info()
_NC, _NS = _info.num_cores, _info.num_subcores
_NW = _NC * _NS


def _make_lookup(B: int, V: int, D: int):
    b_per_w = B // _NW
    mesh = plsc.VectorSubcoreMesh(core_axis_name="c", subcore_axis_name="s")

    @functools.partial(
        pl.kernel,
        mesh=mesh,
        compiler_params=pltpu.CompilerParams(needs_layout_passes=False),
        out_type=jax.ShapeDtypeStruct((B, D), jnp.float32),
        scratch_types=[
            pltpu.VMEM((V, D), jnp.float32),
            pltpu.VMEM((b_per_w,), jnp.int32),
            pltpu.SemaphoreType.DMA,
        ],
    )
    def lookup(table_hbm, idx_hbm, out_hbm, table_v, idx_v, sem):
        wid = lax.axis_index("s") * _NC + lax.axis_index("c")
        base = wid * b_per_w
        pltpu.sync_copy(table_hbm, table_v)
        pltpu.sync_copy(idx_hbm.at[pl.ds(base, b_per_w)], idx_v)

        @pl.loop(0, b_per_w // 16)
        def _(g):
            vec = idx_v[pl.ds(g * 16, 16)]
            for j in range(16):
                row = vec[j]
                pltpu.async_copy(
                    table_v.at[row], out_hbm.at[base + g * 16 + j], sem
                )

        @pl.loop(0, b_per_w)
        def _(r):
            pltpu.make_async_copy(out_hbm.at[0], table_v.at[0], sem).wait()

    return lookup


def kernel(placeholder, table):
    B = placeholder.size
    V, D = table.shape
    idx = placeholder.reshape(B).astype(jnp.int32)
    out = _make_lookup(B, V, D)(table, idx)
    return out.reshape(*placeholder.shape, D)


# speedup vs baseline: 4.9030x; 1.0151x over previous
import functools
import jax
import jax.numpy as jnp
from jax import lax
from jax.experimental import pallas as pl
from jax.experimental.pallas import tpu as pltpu, tpu_sc as plsc

_info = plsc.get_sparse_core_info()
_NC, _NS = _info.num_cores, _info.num_subcores


def _make_lookup(B: int, V: int, D: int, r0: int):
    r1 = B // _NS - r0
    assert r0 % 16 == 0 and r1 % 16 == 0 and _NC == 2
    mesh = plsc.VectorSubcoreMesh(core_axis_name="c", subcore_axis_name="s")

    @functools.partial(
        pl.kernel,
        mesh=mesh,
        compiler_params=pltpu.CompilerParams(needs_layout_passes=False),
        out_type=jax.ShapeDtypeStruct((B, D), jnp.float32),
        scratch_types=[
            pltpu.VMEM((V, D), jnp.float32),
            pltpu.VMEM((max(r0, r1),), jnp.int32),
            pltpu.SemaphoreType.DMA,
        ],
    )
    def lookup(table_hbm, idx_hbm, out_hbm, table_v, idx_v, sem):
        core = lax.axis_index("c")
        sub = lax.axis_index("s")
        pltpu.sync_copy(table_hbm, table_v)

        def run(base, rows):
            pltpu.sync_copy(idx_hbm.at[pl.ds(base, rows)], idx_v.at[pl.ds(0, rows)])

            @pl.loop(0, rows // 16)
            def _(g):
                vec = idx_v[pl.ds(g * 16, 16)]
                for j in range(16):
                    row = vec[j]
                    pltpu.async_copy(
                        table_v.at[row], out_hbm.at[base + g * 16 + j], sem
                    )

            @pl.loop(0, rows)
            def _(r):
                pltpu.make_async_copy(out_hbm.at[0], table_v.at[0], sem).wait()

        @pl.when(core == 0)
        def _():
            run(sub * r0, r0)

        @pl.when(core == 1)
        def _():
            run(_NS * r0 + sub * r1, r1)

    return lookup


def kernel(placeholder, table):
    B = placeholder.size
    V, D = table.shape
    idx = placeholder.reshape(B).astype(jnp.int32)
    out = _make_lookup(B, V, D, r0=976)(table, idx)
    return out.reshape(*placeholder.shape, D)
